# baseline (speedup 1.0000x reference)
import jax
import jax.numpy as jnp
from jax.experimental import pallas as pl
from jax.experimental.pallas import tpu as pltpu

N = 512
D = 64
H = 256
ODIM = 128
R = 10.0
BI = 16
NT = N // BI


def _silu(v):
    return v * jax.nn.sigmoid(v)


def _init_body(ids, x, xt, emb, wc, bc, w1a, w1b, b1,
               dsq_o, en_o, mf_o, h_o, p1_o, p2_o):
    xv = x[:]
    xtv = xt[:]
    s_col = jnp.sum(xv * xv, axis=1, keepdims=True)
    s_row = jnp.sum(xtv * xtv, axis=0, keepdims=True)
    xx = jnp.dot(xv, xtv, preferred_element_type=jnp.float32)
    d2 = jnp.maximum(s_col + s_row - 2.0 * xx, 0.0)
    dist = jnp.sqrt(d2)
    ii = jax.lax.broadcasted_iota(jnp.int32, (N, N), 0)
    jj = jax.lax.broadcasted_iota(jnp.int32, (N, N), 1)
    m = (dist < R) & (dist > 0.0) & (ii != jj)
    dsq_o[:] = d2
    en_o[:] = dist
    mf_o[:] = m.astype(jnp.float32)
    cls = jax.lax.broadcasted_iota(jnp.int32, (N, 32), 1)
    oh = (ids[:] == cls).astype(jnp.float32)
    h = (jnp.dot(oh, emb[:], preferred_element_type=jnp.float32)
         + jnp.dot(xv, wc[:], preferred_element_type=jnp.float32) + bc[:])
    h_o[:] = h
    p1_o[:] = jnp.dot(h, w1a[:], preferred_element_type=jnp.float32) + b1[:]
    p2_o[:] = jnp.dot(h, w1b[:], preferred_element_type=jnp.float32)


def _pair_tile(p1, p2, q, wd, we, w2, b2, wa, ba):
    qv = q[:]
    bf = jnp.bfloat16
    z = (p1[:].astype(bf)[:, None, :]
         + p2[:].astype(bf)[None, :, :]).reshape(BI * N, H)
    z = z + qv[:, 0:1].astype(bf) * wd[:].astype(bf) \
          + qv[:, 1:2].astype(bf) * we[:].astype(bf)
    m1 = z * jax.nn.sigmoid(z)
    t = (jnp.dot(m1, w2[:].astype(bf), preferred_element_type=jnp.float32)
         + b2[:]).astype(bf)
    m2 = t * jax.nn.sigmoid(t)
    a = jax.nn.sigmoid(jnp.dot(m2, wa[:].astype(bf),
                               preferred_element_type=jnp.float32)
                       + ba[:])
    msg = m2.astype(jnp.float32) * (a * qv[:, 2:3])
    return jnp.sum(msg.reshape(BI, N, H), axis=1)


def _node_update(h, aggv, wn1h, wn1a, bn1, wn2, bn2, g, b):
    t = _silu(jnp.dot(h[:], wn1h[:], preferred_element_type=jnp.float32)
              + jnp.dot(aggv, wn1a[:], preferred_element_type=jnp.float32)
              + bn1[:])
    r = h[:] + jnp.dot(t, wn2[:], preferred_element_type=jnp.float32) + bn2[:]
    mu = jnp.mean(r, axis=1, keepdims=True)
    var = jnp.mean((r - mu) ** 2, axis=1, keepdims=True)
    return (r - mu) / jnp.sqrt(var + 1e-5) * g[:] + b[:]


def _pairnode_body(p1, p2, q, wd, we, w2, b2, wa, ba,
                   h, wn1h, wn1a, bn1, wn2, bn2, g, b, w1a, w1b, b1,
                   h_o, p1_o, p2_o, aggs):
    i = pl.program_id(0)
    aggs[pl.ds(i * BI, BI), :] = _pair_tile(p1, p2, q, wd, we, w2, b2, wa, ba)

    @pl.when(i == NT - 1)
    def _():
        hln = _node_update(h, aggs[:], wn1h, wn1a, bn1, wn2, bn2, g, b)
        h_o[:] = hln
        p1_o[:] = jnp.dot(hln, w1a[:],
                          preferred_element_type=jnp.float32) + b1[:]
        p2_o[:] = jnp.dot(hln, w1b[:], preferred_element_type=jnp.float32)


def _pairpool_body(p1, p2, q, wd, we, w2, b2, wa, ba,
                   h, wn1h, wn1a, bn1, wn2, bn2, g, b, wp, bp,
                   ctx_o, aggs):
    i = pl.program_id(0)
    aggs[pl.ds(i * BI, BI), :] = _pair_tile(p1, p2, q, wd, we, w2, b2, wa, ba)

    @pl.when(i == NT - 1)
    def _():
        hln = _node_update(h, aggs[:], wn1h, wn1a, bn1, wn2, bn2, g, b)
        hm = jnp.mean(hln, axis=0, keepdims=True)
        ctx_o[:] = _silu(jnp.dot(hm, wp[:],
                                 preferred_element_type=jnp.float32) + bp[:])


def _f32(shape):
    return jax.ShapeDtypeStruct(shape, jnp.float32)


def _fused_call(body, p1, p2, q, edge_args, node_args, tail_args,
                out_shape):
    cst = lambda s: pl.BlockSpec(s, lambda i: (0, 0))
    row = lambda s: pl.BlockSpec(s, lambda i: (i, 0))
    return pl.pallas_call(
        body,
        grid=(NT,),
        in_specs=[
            row((BI, H)), cst((N, H)),
            row((BI * N, 3)),
            cst((1, H)), cst((1, H)),
            cst((H, H)), cst((1, H)), cst((H, 1)), cst((1, 1)),
            cst((N, D)),
            cst((D, H)), cst((H, H)), cst((1, H)),
            cst((H, D)), cst((1, D)), cst((1, D)), cst((1, D)),
        ] + [cst(a.shape) for a in tail_args],
        out_specs=tuple(cst(s.shape) for s in out_shape),
        out_shape=tuple(out_shape),
        scratch_shapes=[pltpu.VMEM((N, H), jnp.float32)],
    )(p1, p2, q, *edge_args, *node_args, *tail_args)


def kernel(aa_ids, ca_coords, params):
    x = jnp.pad(ca_coords.astype(jnp.float32), ((0, 0), (0, 5)))
    xt = x.T
    ids = aa_ids.astype(jnp.int32).reshape(N, 1)
    emb = jnp.pad(params["aa_embed"], ((0, 11), (0, 0)))
    wc = jnp.pad(params["coord_proj"]["w"], ((0, 5), (0, 0)))
    bc = params["coord_proj"]["b"].reshape(1, D)

    def edge_parts(lp):
        w1 = lp["edge1"]["w"]
        return (w1[0:D], w1[D:2 * D], w1[2 * D:2 * D + 1],
                w1[2 * D + 1:2 * D + 2], lp["edge1"]["b"].reshape(1, H))

    L0, L1 = params["layers"]
    w1a0, w1b0, wd0, we0, b10 = edge_parts(L0)
    w1a1, w1b1, wd1, we1, b11 = edge_parts(L1)

    dsq, en, mf, h0, p1, p2 = pl.pallas_call(
        _init_body,
        out_shape=(_f32((N, N)), _f32((N, N)), _f32((N, N)),
                   _f32((N, D)), _f32((N, H)), _f32((N, H))),
    )(ids, x, xt, emb, wc, bc, w1a0, w1b0, b10)

    q = jnp.concatenate([dsq.reshape(N * N, 1), en.reshape(N * N, 1),
                         mf.reshape(N * N, 1)], axis=1)

    def node_parts(lp):
        wn1 = lp["node1"]["w"]
        return (wn1[0:D], wn1[D:], lp["node1"]["b"].reshape(1, H),
                lp["node2"]["w"], lp["node2"]["b"].reshape(1, D),
                lp["ln_g"].reshape(1, D), lp["ln_b"].reshape(1, D))

    wn1h0, wn1a0, bn10, wn20, bn20, g0, be0 = node_parts(L0)
    h1, p1b, p2b = _fused_call(
        _pairnode_body, p1, p2, q,
        (wd0, we0, L0["edge2"]["w"], L0["edge2"]["b"].reshape(1, H),
         L0["att"]["w"], L0["att"]["b"].reshape(1, 1)),
        (h0, wn1h0, wn1a0, bn10, wn20, bn20, g0, be0),
        (w1a1, w1b1, b11),
        (_f32((N, D)), _f32((N, H)), _f32((N, H))))

    wn1h1, wn1a1, bn11, wn21, bn21, g1, be1 = node_parts(L1)
    ctx = _fused_call(
        _pairpool_body, p1b, p2b, q,
        (wd1, we1, L1["edge2"]["w"], L1["edge2"]["b"].reshape(1, H),
         L1["att"]["w"], L1["att"]["b"].reshape(1, 1)),
        (h1, wn1h1, wn1a1, bn11, wn21, bn21, g1, be1),
        (params["pool"]["w"], params["pool"]["b"].reshape(1, ODIM)),
        (_f32((1, ODIM)),))[0]

    return ctx.reshape(ODIM)


# speedup vs baseline: 1.0004x; 1.0004x over previous
import jax
import jax.numpy as jnp
from jax.experimental import pallas as pl
from jax.experimental.pallas import tpu as pltpu

N = 512
D = 64
H = 256
ODIM = 128
R = 10.0
BI = 32
NT = N // BI


def _silu(v):
    return v * jax.nn.sigmoid(v)


def _init_body(ids, x, xt, emb, wc, bc, w1a, w1b, b1,
               dsq_o, en_o, mf_o, h_o, p1_o, p2_o):
    xv = x[:]
    xtv = xt[:]
    s_col = jnp.sum(xv * xv, axis=1, keepdims=True)
    s_row = jnp.sum(xtv * xtv, axis=0, keepdims=True)
    xx = jnp.dot(xv, xtv, preferred_element_type=jnp.float32)
    d2 = jnp.maximum(s_col + s_row - 2.0 * xx, 0.0)
    dist = jnp.sqrt(d2)
    ii = jax.lax.broadcasted_iota(jnp.int32, (N, N), 0)
    jj = jax.lax.broadcasted_iota(jnp.int32, (N, N), 1)
    m = (dist < R) & (dist > 0.0) & (ii != jj)
    dsq_o[:] = d2
    en_o[:] = dist
    mf_o[:] = m.astype(jnp.float32)
    cls = jax.lax.broadcasted_iota(jnp.int32, (N, 32), 1)
    oh = (ids[:] == cls).astype(jnp.float32)
    h = (jnp.dot(oh, emb[:], preferred_element_type=jnp.float32)
         + jnp.dot(xv, wc[:], preferred_element_type=jnp.float32) + bc[:])
    h_o[:] = h
    p1_o[:] = jnp.dot(h, w1a[:], preferred_element_type=jnp.float32) + b1[:]
    p2_o[:] = jnp.dot(h, w1b[:], preferred_element_type=jnp.float32)


def _pair_tile(p1, p2, q, wd, we, w2, b2, wa, ba):
    qv = q[:]
    bf = jnp.bfloat16
    z = (p1[:].astype(bf)[:, None, :]
         + p2[:].astype(bf)[None, :, :]).reshape(BI * N, H)
    z = z + qv[:, 0:1].astype(bf) * wd[:].astype(bf) \
          + qv[:, 1:2].astype(bf) * we[:].astype(bf)
    m1 = z * jax.nn.sigmoid(z)
    t = (jnp.dot(m1, w2[:].astype(bf), preferred_element_type=jnp.float32)
         + b2[:]).astype(bf)
    m2 = t * jax.nn.sigmoid(t)
    a = jax.nn.sigmoid(jnp.dot(m2, wa[:].astype(bf),
                               preferred_element_type=jnp.float32)
                       + ba[:])
    msg = m2.astype(jnp.float32) * (a * qv[:, 2:3])
    return jnp.sum(msg.reshape(BI, N, H), axis=1)


def _node_update(h, aggv, wn1h, wn1a, bn1, wn2, bn2, g, b):
    t = _silu(jnp.dot(h[:], wn1h[:], preferred_element_type=jnp.float32)
              + jnp.dot(aggv, wn1a[:], preferred_element_type=jnp.float32)
              + bn1[:])
    r = h[:] + jnp.dot(t, wn2[:], preferred_element_type=jnp.float32) + bn2[:]
    mu = jnp.mean(r, axis=1, keepdims=True)
    var = jnp.mean((r - mu) ** 2, axis=1, keepdims=True)
    return (r - mu) / jnp.sqrt(var + 1e-5) * g[:] + b[:]


def _pairnode_body(p1, p2, q, wd, we, w2, b2, wa, ba,
                   h, wn1h, wn1a, bn1, wn2, bn2, g, b, w1a, w1b, b1,
                   h_o, p1_o, p2_o, aggs):
    i = pl.program_id(0)
    aggs[pl.ds(i * BI, BI), :] = _pair_tile(p1, p2, q, wd, we, w2, b2, wa, ba)

    @pl.when(i == NT - 1)
    def _():
        hln = _node_update(h, aggs[:], wn1h, wn1a, bn1, wn2, bn2, g, b)
        h_o[:] = hln
        p1_o[:] = jnp.dot(hln, w1a[:],
                          preferred_element_type=jnp.float32) + b1[:]
        p2_o[:] = jnp.dot(hln, w1b[:], preferred_element_type=jnp.float32)


def _pairpool_body(p1, p2, q, wd, we, w2, b2, wa, ba,
                   h, wn1h, wn1a, bn1, wn2, bn2, g, b, wp, bp,
                   ctx_o, aggs):
    i = pl.program_id(0)
    aggs[pl.ds(i * BI, BI), :] = _pair_tile(p1, p2, q, wd, we, w2, b2, wa, ba)

    @pl.when(i == NT - 1)
    def _():
        hln = _node_update(h, aggs[:], wn1h, wn1a, bn1, wn2, bn2, g, b)
        hm = jnp.mean(hln, axis=0, keepdims=True)
        ctx_o[:] = _silu(jnp.dot(hm, wp[:],
                                 preferred_element_type=jnp.float32) + bp[:])


def _f32(shape):
    return jax.ShapeDtypeStruct(shape, jnp.float32)


def _fused_call(body, p1, p2, q, edge_args, node_args, tail_args,
                out_shape):
    cst = lambda s: pl.BlockSpec(s, lambda i: (0, 0))
    row = lambda s: pl.BlockSpec(s, lambda i: (i, 0))
    return pl.pallas_call(
        body,
        grid=(NT,),
        in_specs=[
            row((BI, H)), cst((N, H)),
            row((BI * N, 3)),
            cst((1, H)), cst((1, H)),
            cst((H, H)), cst((1, H)), cst((H, 1)), cst((1, 1)),
            cst((N, D)),
            cst((D, H)), cst((H, H)), cst((1, H)),
            cst((H, D)), cst((1, D)), cst((1, D)), cst((1, D)),
        ] + [cst(a.shape) for a in tail_args],
        out_specs=tuple(cst(s.shape) for s in out_shape),
        out_shape=tuple(out_shape),
        scratch_shapes=[pltpu.VMEM((N, H), jnp.float32)],
    )(p1, p2, q, *edge_args, *node_args, *tail_args)


def kernel(aa_ids, ca_coords, params):
    x = jnp.pad(ca_coords.astype(jnp.float32), ((0, 0), (0, 5)))
    xt = x.T
    ids = aa_ids.astype(jnp.int32).reshape(N, 1)
    emb = jnp.pad(params["aa_embed"], ((0, 11), (0, 0)))
    wc = jnp.pad(params["coord_proj"]["w"], ((0, 5), (0, 0)))
    bc = params["coord_proj"]["b"].reshape(1, D)

    def edge_parts(lp):
        w1 = lp["edge1"]["w"]
        return (w1[0:D], w1[D:2 * D], w1[2 * D:2 * D + 1],
                w1[2 * D + 1:2 * D + 2], lp["edge1"]["b"].reshape(1, H))

    L0, L1 = params["layers"]
    w1a0, w1b0, wd0, we0, b10 = edge_parts(L0)
    w1a1, w1b1, wd1, we1, b11 = edge_parts(L1)

    dsq, en, mf, h0, p1, p2 = pl.pallas_call(
        _init_body,
        out_shape=(_f32((N, N)), _f32((N, N)), _f32((N, N)),
                   _f32((N, D)), _f32((N, H)), _f32((N, H))),
    )(ids, x, xt, emb, wc, bc, w1a0, w1b0, b10)

    q = jnp.concatenate([dsq.reshape(N * N, 1), en.reshape(N * N, 1),
                         mf.reshape(N * N, 1)], axis=1)

    def node_parts(lp):
        wn1 = lp["node1"]["w"]
        return (wn1[0:D], wn1[D:], lp["node1"]["b"].reshape(1, H),
                lp["node2"]["w"], lp["node2"]["b"].reshape(1, D),
                lp["ln_g"].reshape(1, D), lp["ln_b"].reshape(1, D))

    wn1h0, wn1a0, bn10, wn20, bn20, g0, be0 = node_parts(L0)
    h1, p1b, p2b = _fused_call(
        _pairnode_body, p1, p2, q,
        (wd0, we0, L0["edge2"]["w"], L0["edge2"]["b"].reshape(1, H),
         L0["att"]["w"], L0["att"]["b"].reshape(1, 1)),
        (h0, wn1h0, wn1a0, bn10, wn20, bn20, g0, be0),
        (w1a1, w1b1, b11),
        (_f32((N, D)), _f32((N, H)), _f32((N, H))))

    wn1h1, wn1a1, bn11, wn21, bn21, g1, be1 = node_parts(L1)
    ctx = _fused_call(
        _pairpool_body, p1b, p2b, q,
        (wd1, we1, L1["edge2"]["w"], L1["edge2"]["b"].reshape(1, H),
         L1["att"]["w"], L1["att"]["b"].reshape(1, 1)),
        (h1, wn1h1, wn1a1, bn11, wn21, bn21, g1, be1),
        (params["pool"]["w"], params["pool"]["b"].reshape(1, ODIM)),
        (_f32((1, ODIM)),))[0]

    return ctx.reshape(ODIM)
